# Initial kernel scaffold; baseline (speedup 1.0000x reference)
import functools
import math

import jax
import jax.numpy as jnp
from jax import lax
from jax.experimental import pallas as pl
from jax.experimental.pallas import tpu as pltpu
from jax.experimental.pallas import tpu_sc as plsc

_N = 20000
_C = 81
_CF = 80
_K = 200
_DETS = 100
_IMG_W = 1333.0
_IMG_H = 800.0
_SCORE_T = 0.05
_NMS_T = 0.5
_CLIP = math.log(1000.0 / 16.0)

_NW = 32
_PW = 512
_A_BLK = 2000


def _softmax_body(logit_ref, out_ref):
    x = logit_ref[...]
    m = jnp.max(x, axis=1, keepdims=True)
    e = jnp.exp(x - m)
    s = jnp.sum(e, axis=1, keepdims=True)
    p = e / s
    fg = p[:, 1:]
    out_ref[...] = jnp.where(fg > _SCORE_T, fg, -1.0)


def _masked_scores(class_logits):
    return pl.pallas_call(
        _softmax_body,
        grid=(_N // _A_BLK,),
        in_specs=[pl.BlockSpec((_A_BLK, _C), lambda i: (i, 0))],
        out_specs=pl.BlockSpec((_A_BLK, _CF), lambda i: (i, 0)),
        out_shape=jax.ShapeDtypeStruct((_N, _CF), jnp.float32),
    )(class_logits)


def _gather_body(reg_hbm, prop_hbm, ridx_hbm, pidx_hbm, oreg_hbm, oprop_hbm,
                 idxr_v, idxp_v, regrows_v, proprows_v, sem):
    c = lax.axis_index("c")
    s = lax.axis_index("s")
    w = s * 2 + c
    pltpu.sync_copy(ridx_hbm.at[w], idxr_v)
    pltpu.sync_copy(pidx_hbm.at[w], idxp_v)
    copies = []
    for j in range(_PW // 128):
        cp = pltpu.make_async_copy(
            reg_hbm.at[idxr_v.at[j]], regrows_v.at[pl.ds(j * 128, 128)], sem)
        cp.start()
        copies.append(cp)
    for j in range(_PW // 128):
        cp = pltpu.make_async_copy(
            prop_hbm.at[idxp_v.at[j]], proprows_v.at[pl.ds(j * 128, 128)], sem)
        cp.start()
        copies.append(cp)
    for cp in copies:
        cp.wait()
    pltpu.sync_copy(regrows_v, oreg_hbm.at[w])
    pltpu.sync_copy(proprows_v, oprop_hbm.at[w])


def _gather_candidates(reg_flat, props, ridx, pidx):
    mesh = plsc.VectorSubcoreMesh(core_axis_name="c", subcore_axis_name="s")
    fn = functools.partial(
        pl.kernel,
        mesh=mesh,
        out_type=[
            jax.ShapeDtypeStruct((_NW, _PW, 4), jnp.float32),
            jax.ShapeDtypeStruct((_NW, _PW, 4), jnp.float32),
        ],
        scratch_types=[
            pltpu.VMEM((_PW // 128, 128), jnp.int32),
            pltpu.VMEM((_PW // 128, 128), jnp.int32),
            pltpu.VMEM((_PW, 4), jnp.float32),
            pltpu.VMEM((_PW, 4), jnp.float32),
            pltpu.SemaphoreType.DMA,
        ],
    )(_gather_body)
    return fn(reg_flat, props, ridx, pidx)


def _nms_body(sc_ref, reg_ref, prop_ref, outs_ref, outb_ref, area_ref, keep_ref):
    px1 = prop_ref[0]
    py1 = prop_ref[1]
    px2 = prop_ref[2]
    py2 = prop_ref[3]
    widths = px2 - px1 + 1.0
    heights = py2 - py1 + 1.0
    ctr_x = px1 + 0.5 * widths
    ctr_y = py1 + 0.5 * heights
    dx = reg_ref[0] / 10.0
    dy = reg_ref[1] / 10.0
    dw = jnp.minimum(reg_ref[2] / 5.0, _CLIP)
    dh = jnp.minimum(reg_ref[3] / 5.0, _CLIP)
    pred_ctr_x = dx * widths + ctr_x
    pred_ctr_y = dy * heights + ctr_y
    pred_w = jnp.exp(dw) * widths
    pred_h = jnp.exp(dh) * heights
    x1 = jnp.clip(pred_ctr_x - 0.5 * pred_w, 0.0, _IMG_W - 1.0)
    y1 = jnp.clip(pred_ctr_y - 0.5 * pred_h, 0.0, _IMG_H - 1.0)
    x2 = jnp.clip(pred_ctr_x + 0.5 * pred_w - 1.0, 0.0, _IMG_W - 1.0)
    y2 = jnp.clip(pred_ctr_y + 0.5 * pred_h - 1.0, 0.0, _IMG_H - 1.0)
    outb_ref[0] = x1
    outb_ref[1] = y1
    outb_ref[2] = x2
    outb_ref[3] = y2
    area_ref[...] = (x2 - x1 + 1.0) * (y2 - y1 + 1.0)
    area = area_ref[...]
    sc = sc_ref[...]
    keep_ref[...] = jnp.where(sc > _SCORE_T, 1.0, 0.0)
    row = lax.broadcasted_iota(jnp.int32, (_K, _CF), 0)

    def body(i, carry):
        a1 = outb_ref[0, pl.ds(i, 1), :]
        b1 = outb_ref[1, pl.ds(i, 1), :]
        a2 = outb_ref[2, pl.ds(i, 1), :]
        b2 = outb_ref[3, pl.ds(i, 1), :]
        ai = area_ref[pl.ds(i, 1), :]
        ltx = jnp.maximum(x1, a1)
        lty = jnp.maximum(y1, b1)
        rbx = jnp.minimum(x2, a2)
        rby = jnp.minimum(y2, b2)
        w = jnp.maximum(rbx - ltx + 1.0, 0.0)
        h = jnp.maximum(rby - lty + 1.0, 0.0)
        inter = w * h
        iou = inter / (area + ai - inter)
        keep = keep_ref[...]
        earlier = (keep > 0.5) & (row < i)
        sup = jnp.any((iou > _NMS_T) & earlier, axis=0, keepdims=True)
        ki = keep_ref[pl.ds(i, 1), :]
        keep_ref[pl.ds(i, 1), :] = jnp.where(sup, 0.0, ki)
        return carry

    lax.fori_loop(1, _K, body, 0)
    outs_ref[...] = jnp.where(keep_ref[...] > 0.5, sc, -1.0)


def _nms(sc_t, creg, cprop):
    return pl.pallas_call(
        _nms_body,
        out_shape=[
            jax.ShapeDtypeStruct((_K, _CF), jnp.float32),
            jax.ShapeDtypeStruct((4, _K, _CF), jnp.float32),
        ],
        scratch_shapes=[
            pltpu.VMEM((_K, _CF), jnp.float32),
            pltpu.VMEM((_K, _CF), jnp.float32),
        ],
    )(sc_t, creg, cprop)


@jax.jit
def kernel(class_logits, box_regression, proposals):
    masked = _masked_scores(class_logits)
    masked_t = masked.T
    top_scores, top_idx = lax.top_k(masked_t, _K)

    cls = jnp.arange(1, _C, dtype=jnp.int32)[:, None]
    rrows = top_idx * _C + cls
    pad_n = _NW * _PW - _CF * _K
    pad_p = (jnp.arange(pad_n, dtype=jnp.int32) * 37) % _N
    ridx = jnp.concatenate([rrows.reshape(-1), pad_p * _C]).reshape(_NW, _PW // 128, 128)
    pidx = jnp.concatenate([top_idx.reshape(-1), pad_p]).reshape(_NW, _PW // 128, 128)

    reg_flat = box_regression.reshape(_N * _C, 4)
    oreg, oprop = _gather_candidates(reg_flat, proposals, ridx, pidx)

    creg = oreg.reshape(_NW * _PW, 4)[: _CF * _K]
    creg = creg.reshape(_CF, _K, 4).transpose(2, 1, 0)
    cprop = oprop.reshape(_NW * _PW, 4)[: _CF * _K]
    cprop = cprop.reshape(_CF, _K, 4).transpose(2, 1, 0)
    sc_t = top_scores.T

    outs, outb = _nms(sc_t, creg, cprop)

    flat_scores = outs.T.reshape(-1)
    flat_boxes = outb.transpose(2, 1, 0).reshape(_CF * _K, 4)
    fs, fi = lax.top_k(flat_scores, _DETS)
    top_boxes = flat_boxes[fi]
    top_labels = (fi // _K + 1).astype(jnp.int32)
    return top_boxes, fs, top_labels


# speedup vs baseline: 1.5508x; 1.5508x over previous
import functools
import math

import jax
import jax.numpy as jnp
from jax import lax
from jax.experimental import pallas as pl
from jax.experimental.pallas import tpu as pltpu
from jax.experimental.pallas import tpu_sc as plsc

_N = 20000
_C = 81
_CF = 80
_K = 200
_DETS = 100
_IMG_W = 1333.0
_IMG_H = 800.0
_SCORE_T = 0.05
_NMS_T = 0.5
_CLIP = math.log(1000.0 / 16.0)

_NW = 32
_PW = 512
_A_BLK = 2000


def _softmax_body(logit_ref, out_ref):
    x = logit_ref[...]
    m = jnp.max(x, axis=1, keepdims=True)
    e = jnp.exp(x - m)
    s = jnp.sum(e, axis=1, keepdims=True)
    p = e / s
    fg = p[:, 1:]
    out_ref[...] = jnp.where(fg > _SCORE_T, fg, -1.0)


def _masked_scores(class_logits):
    return pl.pallas_call(
        _softmax_body,
        grid=(_N // _A_BLK,),
        in_specs=[pl.BlockSpec((_A_BLK, _C), lambda i: (i, 0))],
        out_specs=pl.BlockSpec((_A_BLK, _CF), lambda i: (i, 0)),
        out_shape=jax.ShapeDtypeStruct((_N, _CF), jnp.float32),
    )(class_logits)


def _gather_body(reg_hbm, prop_hbm, ridx_hbm, pidx_hbm, oreg_hbm, oprop_hbm,
                 idxr_v, idxp_v, regrows_v, proprows_v, sem):
    c = lax.axis_index("c")
    s = lax.axis_index("s")
    w = s * 2 + c
    pltpu.sync_copy(ridx_hbm.at[w], idxr_v)
    pltpu.sync_copy(pidx_hbm.at[w], idxp_v)
    copies = []
    for ch in range(4):
        for j in range(_PW // 128):
            cp = pltpu.make_async_copy(
                reg_hbm.at[idxr_v.at[ch, j]],
                regrows_v.at[ch, pl.ds(j * 128, 128)], sem)
            cp.start()
            copies.append(cp)
            cp = pltpu.make_async_copy(
                prop_hbm.at[idxp_v.at[ch, j]],
                proprows_v.at[ch, pl.ds(j * 128, 128)], sem)
            cp.start()
            copies.append(cp)
    for cp in copies:
        cp.wait()
    pltpu.sync_copy(regrows_v, oreg_hbm.at[w])
    pltpu.sync_copy(proprows_v, oprop_hbm.at[w])


def _gather_candidates(reg_flat, props, ridx, pidx):
    mesh = plsc.VectorSubcoreMesh(core_axis_name="c", subcore_axis_name="s")
    fn = functools.partial(
        pl.kernel,
        mesh=mesh,
        out_type=[
            jax.ShapeDtypeStruct((_NW, 4, _PW), jnp.float32),
            jax.ShapeDtypeStruct((_NW, 4, _PW), jnp.float32),
        ],
        scratch_types=[
            pltpu.VMEM((4, _PW // 128, 128), jnp.int32),
            pltpu.VMEM((4, _PW // 128, 128), jnp.int32),
            pltpu.VMEM((4, _PW), jnp.float32),
            pltpu.VMEM((4, _PW), jnp.float32),
            pltpu.SemaphoreType.DMA,
        ],
    )(_gather_body)
    return fn(reg_flat, props, ridx, pidx)


def _nms_body(sc_ref, reg_ref, prop_ref, outs_ref, outb_ref, area_ref, keep_ref):
    px1 = prop_ref[0]
    py1 = prop_ref[1]
    px2 = prop_ref[2]
    py2 = prop_ref[3]
    widths = px2 - px1 + 1.0
    heights = py2 - py1 + 1.0
    ctr_x = px1 + 0.5 * widths
    ctr_y = py1 + 0.5 * heights
    dx = reg_ref[0] / 10.0
    dy = reg_ref[1] / 10.0
    dw = jnp.minimum(reg_ref[2] / 5.0, _CLIP)
    dh = jnp.minimum(reg_ref[3] / 5.0, _CLIP)
    pred_ctr_x = dx * widths + ctr_x
    pred_ctr_y = dy * heights + ctr_y
    pred_w = jnp.exp(dw) * widths
    pred_h = jnp.exp(dh) * heights
    x1 = jnp.clip(pred_ctr_x - 0.5 * pred_w, 0.0, _IMG_W - 1.0)
    y1 = jnp.clip(pred_ctr_y - 0.5 * pred_h, 0.0, _IMG_H - 1.0)
    x2 = jnp.clip(pred_ctr_x + 0.5 * pred_w - 1.0, 0.0, _IMG_W - 1.0)
    y2 = jnp.clip(pred_ctr_y + 0.5 * pred_h - 1.0, 0.0, _IMG_H - 1.0)
    outb_ref[0] = x1
    outb_ref[1] = y1
    outb_ref[2] = x2
    outb_ref[3] = y2
    area_ref[...] = (x2 - x1 + 1.0) * (y2 - y1 + 1.0)
    area = area_ref[...]
    sc = sc_ref[...]
    keep_ref[...] = jnp.where(sc > _SCORE_T, 1.0, 0.0)
    row = lax.broadcasted_iota(jnp.int32, (_K, _CF), 0)

    def body(i, carry):
        a1 = outb_ref[0, pl.ds(i, 1), :]
        b1 = outb_ref[1, pl.ds(i, 1), :]
        a2 = outb_ref[2, pl.ds(i, 1), :]
        b2 = outb_ref[3, pl.ds(i, 1), :]
        ai = area_ref[pl.ds(i, 1), :]
        ltx = jnp.maximum(x1, a1)
        lty = jnp.maximum(y1, b1)
        rbx = jnp.minimum(x2, a2)
        rby = jnp.minimum(y2, b2)
        w = jnp.maximum(rbx - ltx + 1.0, 0.0)
        h = jnp.maximum(rby - lty + 1.0, 0.0)
        inter = w * h
        iou = inter / (area + ai - inter)
        keep = keep_ref[...]
        earlier = (keep > 0.5) & (row < i)
        sup = jnp.any((iou > _NMS_T) & earlier, axis=0, keepdims=True)
        ki = keep_ref[pl.ds(i, 1), :]
        keep_ref[pl.ds(i, 1), :] = jnp.where(sup, 0.0, ki)
        return carry

    lax.fori_loop(1, _K, body, 0)
    outs_ref[...] = jnp.where(keep_ref[...] > 0.5, sc, -1.0)


def _nms(sc_t, creg, cprop):
    return pl.pallas_call(
        _nms_body,
        out_shape=[
            jax.ShapeDtypeStruct((_K, _CF), jnp.float32),
            jax.ShapeDtypeStruct((4, _K, _CF), jnp.float32),
        ],
        scratch_shapes=[
            pltpu.VMEM((_K, _CF), jnp.float32),
            pltpu.VMEM((_K, _CF), jnp.float32),
        ],
    )(sc_t, creg, cprop)


@jax.jit
def kernel(class_logits, box_regression, proposals):
    masked = _masked_scores(class_logits)
    masked_t = masked.T
    top_scores, top_idx = lax.top_k(masked_t, _K)

    cls = jnp.arange(1, _C, dtype=jnp.int32)[:, None]
    rrows = top_idx * _C + cls
    pad_n = _NW * _PW - _CF * _K
    pad_p = (jnp.arange(pad_n, dtype=jnp.int32) * 37) % _N
    rflat = jnp.concatenate([rrows.reshape(-1), pad_p * _C])
    pflat = jnp.concatenate([top_idx.reshape(-1), pad_p])
    ch_off = jnp.arange(4, dtype=jnp.int32)[:, None]
    ridx = (rflat[None, :] * 4 + ch_off).reshape(4, _NW, _PW // 128, 128)
    ridx = ridx.transpose(1, 0, 2, 3)
    pidx = (pflat[None, :] * 4 + ch_off).reshape(4, _NW, _PW // 128, 128)
    pidx = pidx.transpose(1, 0, 2, 3)

    reg_1d = box_regression.reshape(_N * _C * 4)
    prop_1d = proposals.reshape(_N * 4)
    oreg, oprop = _gather_candidates(reg_1d, prop_1d, ridx, pidx)

    creg = oreg.transpose(1, 0, 2).reshape(4, _NW * _PW)[:, : _CF * _K]
    creg = creg.reshape(4, _CF, _K).transpose(0, 2, 1)
    cprop = oprop.transpose(1, 0, 2).reshape(4, _NW * _PW)[:, : _CF * _K]
    cprop = cprop.reshape(4, _CF, _K).transpose(0, 2, 1)
    sc_t = top_scores.T

    outs, outb = _nms(sc_t, creg, cprop)

    flat_scores = outs.T.reshape(-1)
    flat_boxes = outb.transpose(2, 1, 0).reshape(_CF * _K, 4)
    fs, fi = lax.top_k(flat_scores, _DETS)
    top_boxes = flat_boxes[fi]
    top_labels = (fi // _K + 1).astype(jnp.int32)
    return top_boxes, fs, top_labels
